# baseline (speedup 1.0000x reference)
import functools

import jax
import jax.numpy as jnp
from jax import lax
from jax.experimental import pallas as pl
from jax.experimental.pallas import tpu as pltpu
from jax.experimental.pallas import tpu_sc as plsc

NC = 2
NS = 16
L = 16
CHUNK = 128
RB = 8


@functools.lru_cache(maxsize=None)
def _convert(N, D):
    NW = NC * NS
    G = (N * D) // CHUNK
    NBLK = N // CHUNK
    NRND = (NBLK + NW * RB - 1) // (NW * RB)
    TAIL = N - NBLK * CHUNK

    mesh = plsc.VectorSubcoreMesh(
        core_axis_name="c", subcore_axis_name="s",
        num_cores=NC, num_subcores=NS)

    out_t = jax.ShapeDtypeStruct((G, CHUNK), jnp.float32)

    @functools.partial(
        pl.kernel,
        out_type=(out_t, out_t, out_t, out_t),
        mesh=mesh,
        compiler_params=pltpu.CompilerParams(needs_layout_passes=False),
        scratch_types=[
            pltpu.VMEM((D, RB * CHUNK), jnp.float32),
            pltpu.VMEM((RB * L, CHUNK), jnp.float32),
            pltpu.VMEM((TAIL, D), jnp.float32),
        ],
    )
    def convert_kernel(t0_hbm, t1_hbm, t2_hbm, t3_hbm,
                       e0_hbm, e1_hbm, e2_hbm, e3_hbm,
                       o0_hbm, o1_hbm, o2_hbm, o3_hbm,
                       in_v, out_v, tail_v):
        wid = lax.axis_index("s") * NC + lax.axis_index("c")
        iota = lax.iota(jnp.int32, L)

        def transpose_round(blk0, nb, tbl, out):
            nu = nb * CHUNK
            pltpu.sync_copy(tbl.at[:, pl.ds(blk0 * CHUNK, nu)],
                            in_v.at[:, pl.ds(0, nu)])

            def col_body(q, carry2):
                for l in range(L):
                    ul = q * L + l
                    vec = plsc.load_gather(
                        in_v, [iota, jnp.full((L,), ul, jnp.int32)])
                    plsc.store_scatter(
                        out_v,
                        [jnp.full((L,), ul // 8, jnp.int32),
                         (ul % 8) * L + iota],
                        vec)
                return carry2

            lax.fori_loop(0, nb * (CHUNK // L), col_body, 0, unroll=False)
            pltpu.sync_copy(out_v.at[pl.ds(0, nb * L), :],
                            out.at[pl.ds(blk0 * L, nb * L), :])

        NFULL = NBLK // RB * RB
        for tbl, out in ((t0_hbm, o0_hbm), (t1_hbm, o1_hbm),
                         (t2_hbm, o2_hbm), (t3_hbm, o3_hbm)):
            def round_body(r, carry, tbl=tbl, out=out):
                blk0 = (r * NW + wid) * RB

                @pl.when(blk0 < NFULL)
                def _():
                    transpose_round(blk0, RB, tbl, out)
                return carry

            lax.fori_loop(0, NRND, round_body, 0, unroll=False)

            @pl.when(wid < NBLK - NFULL)
            def _(tbl=tbl, out=out):
                transpose_round(NFULL + wid, 1, tbl, out)

        @pl.when(wid == 0)
        def _():
            for tail, out in ((e0_hbm, o0_hbm), (e1_hbm, o1_hbm),
                              (e2_hbm, o2_hbm), (e3_hbm, o3_hbm)):
                pltpu.sync_copy(tail, tail_v)
                for p in range(TAIL // 8):
                    for q in range(8):
                        vec = plsc.load_gather(
                            tail_v,
                            [jnp.full((L,), 8 * p + q, jnp.int32), iota])
                        plsc.store_scatter(
                            out_v,
                            [jnp.full((L,), p, jnp.int32), q * L + iota],
                            vec)
                pltpu.sync_copy(
                    out_v.at[pl.ds(0, TAIL // 8)],
                    out.at[pl.ds(G - TAIL // 8, TAIL // 8), :])

    return convert_kernel


@functools.lru_cache(maxsize=None)
def _gather(B, D):
    assert D == L
    NW = NC * NS
    BPW = B // NW
    assert BPW % CHUNK == 0
    NCH = BPW // CHUNK

    mesh = plsc.VectorSubcoreMesh(
        core_axis_name="c", subcore_axis_name="s",
        num_cores=NC, num_subcores=NS)

    @functools.partial(
        pl.kernel,
        out_type=jax.ShapeDtypeStruct((B,), jnp.float32),
        mesh=mesh,
        compiler_params=pltpu.CompilerParams(needs_layout_passes=False),
        scratch_types=[
            pltpu.VMEM((BPW,), jnp.int32),
            pltpu.VMEM((BPW,), jnp.int32),
            pltpu.VMEM((BPW,), jnp.int32),
            pltpu.VMEM((BPW,), jnp.int32),
            pltpu.VMEM((CHUNK, 128), jnp.float32),
            pltpu.VMEM((CHUNK, 128), jnp.float32),
            pltpu.VMEM((CHUNK, 128), jnp.float32),
            pltpu.VMEM((CHUNK, 128), jnp.float32),
            pltpu.VMEM((BPW,), jnp.float32),
            pltpu.VMEM((BPW,), jnp.float32),
            pltpu.SemaphoreType.DMA,
            pltpu.SemaphoreType.DMA,
        ],
    )
    def gather_kernel(u_hbm, i_hbm, mod_u_hbm, mod_i_hbm, mem_u_hbm,
                      mem_i_hbm, alpha_hbm, out_hbm,
                      u_v, i_v, u8_v, i8_v, mu_v, mi_v, ku_v, ki_v,
                      a_v, o_v, sem, sem_a):
        wid = lax.axis_index("s") * NC + lax.axis_index("c")
        base = wid * BPW

        pltpu.sync_copy(u_hbm.at[pl.ds(base, BPW)], u_v)
        pltpu.sync_copy(i_hbm.at[pl.ds(base, BPW)], i_v)

        iota = lax.iota(jnp.int32, L)

        def shift_body(b, carry):
            s = pl.ds(b * L, L)
            u8_v[s] = lax.shift_right_logical(u_v[s], 3)
            i8_v[s] = lax.shift_right_logical(i_v[s], 3)
            return carry

        lax.fori_loop(0, BPW // L, shift_body, 0, unroll=False)

        a_copies = []
        for j in range(NCH):
            s = pl.ds(j * CHUNK, CHUNK)
            a_copies.append(
                pltpu.async_copy(alpha_hbm.at[u_v.at[s]], a_v.at[s], sem_a))
        for c in a_copies:
            c.wait()

        for j in range(NCH):
            s = pl.ds(j * CHUNK, CHUNK)
            copies = [
                pltpu.async_copy(mod_u_hbm.at[u8_v.at[s]], mu_v, sem),
                pltpu.async_copy(mod_i_hbm.at[i8_v.at[s]], mi_v, sem),
                pltpu.async_copy(mem_u_hbm.at[u8_v.at[s]], ku_v, sem),
                pltpu.async_copy(mem_i_hbm.at[i8_v.at[s]], ki_v, sem),
            ]
            for c in copies:
                c.wait()

            def blk_body(b, carry):
                pos = j * CHUNK + b * L
                rows = b * L + iota
                ucol = (u_v[pl.ds(pos, L)] & 7) * L
                icol = (i_v[pl.ds(pos, L)] & 7) * L
                acc1 = jnp.zeros((L,), jnp.float32)
                acc2 = jnp.zeros((L,), jnp.float32)
                for d in range(D):
                    acc1 = acc1 + (plsc.load_gather(mu_v, [rows, ucol + d])
                                   * plsc.load_gather(mi_v, [rows, icol + d]))
                    acc2 = acc2 + (plsc.load_gather(ku_v, [rows, ucol + d])
                                   * plsc.load_gather(ki_v, [rows, icol + d]))
                a = a_v[pl.ds(pos, L)]
                o_v[pl.ds(pos, L)] = a * acc1 + (1.0 - a) * acc2
                return carry

            lax.fori_loop(0, CHUNK // L, blk_body, 0, unroll=False)

        pltpu.sync_copy(o_v, out_hbm.at[pl.ds(base, BPW)])

    return gather_kernel


def kernel(user_indices, item_indices, mod_user_emb, mod_item_emb,
           mem_user_emb, mem_item_emb, alpha_table):
    B = user_indices.shape[0]
    N, D = mod_user_emb.shape
    tables = (mod_user_emb, mod_item_emb, mem_user_emb, mem_item_emb)
    tail0 = (N // 128) * 128
    conv = _convert(N, D)(
        *(t.T for t in tables), *(t[tail0:, :] for t in tables))
    return _gather(B, D)(user_indices, item_indices, *conv,
                         alpha_table.reshape(-1))


# speedup vs baseline: 1.1037x; 1.1037x over previous
import functools

import jax
import jax.numpy as jnp
from jax import lax
from jax.experimental import pallas as pl
from jax.experimental.pallas import tpu as pltpu
from jax.experimental.pallas import tpu_sc as plsc

NC = 2
NS = 16
L = 16
CHUNK = 128
RB = 4


@functools.lru_cache(maxsize=None)
def _convert(N, D):
    NW = NC * NS
    G = (N * D) // CHUNK
    NBLK = N // CHUNK
    NRND = (NBLK + NW * RB - 1) // (NW * RB)
    TAIL = N - NBLK * CHUNK

    mesh = plsc.VectorSubcoreMesh(
        core_axis_name="c", subcore_axis_name="s",
        num_cores=NC, num_subcores=NS)

    out_t = jax.ShapeDtypeStruct((G, CHUNK), jnp.float32)

    @functools.partial(
        pl.kernel,
        out_type=(out_t, out_t, out_t, out_t),
        mesh=mesh,
        compiler_params=pltpu.CompilerParams(needs_layout_passes=False),
        scratch_types=[
            pltpu.VMEM((D, RB * CHUNK), jnp.float32),
            pltpu.VMEM((D, RB * CHUNK), jnp.float32),
            pltpu.VMEM((D, RB * CHUNK), jnp.float32),
            pltpu.VMEM((D, RB * CHUNK), jnp.float32),
            pltpu.VMEM((RB * L, CHUNK), jnp.float32),
            pltpu.VMEM((RB * L, CHUNK), jnp.float32),
            pltpu.VMEM((RB * L, CHUNK), jnp.float32),
            pltpu.VMEM((RB * L, CHUNK), jnp.float32),
            pltpu.VMEM((TAIL, D), jnp.float32),
            pltpu.SemaphoreType.DMA,
            pltpu.SemaphoreType.DMA,
        ],
    )
    def convert_kernel(t0_hbm, t1_hbm, t2_hbm, t3_hbm,
                       e0_hbm, e1_hbm, e2_hbm, e3_hbm,
                       o0_hbm, o1_hbm, o2_hbm, o3_hbm,
                       in_v0, in_v1, in_v2, in_v3,
                       out_v0, out_v1, out_v2, out_v3,
                       tail_v, sem_in, sem_out):
        wid = lax.axis_index("s") * NC + lax.axis_index("c")
        iota = lax.iota(jnp.int32, L)
        in_bufs = (in_v0, in_v1, in_v2, in_v3)
        out_bufs = (out_v0, out_v1, out_v2, out_v3)
        in_v, out_v = in_v0, out_v0

        def transpose_compute(in_b, out_b, nb):
            def col_body(q, carry2):
                for l in range(L):
                    ul = q * L + l
                    vec = plsc.load_gather(
                        in_b, [iota, jnp.full((L,), ul, jnp.int32)])
                    plsc.store_scatter(
                        out_b,
                        [jnp.full((L,), ul // 8, jnp.int32),
                         (ul % 8) * L + iota],
                        vec)
                return carry2

            lax.fori_loop(0, nb * (CHUNK // L), col_body, 0, unroll=False)

        tables = (t0_hbm, t1_hbm, t2_hbm, t3_hbm)
        outs = (o0_hbm, o1_hbm, o2_hbm, o3_hbm)
        NFULL = NBLK // RB * RB

        def round_body(r, carry):
            blk0 = (r * NW + wid) * RB

            @pl.when(blk0 < NFULL)
            def _():
                src = pl.ds(blk0 * CHUNK, RB * CHUNK)
                ins = [pltpu.async_copy(tables[k].at[:, src], in_bufs[k],
                                        sem_in)
                       for k in range(4)]
                ocs = []
                for k in range(4):
                    ins[k].wait()
                    transpose_compute(in_bufs[k], out_bufs[k], RB)
                    ocs.append(pltpu.async_copy(
                        out_bufs[k],
                        outs[k].at[pl.ds(blk0 * L, RB * L), :], sem_out))
                for c in ocs:
                    c.wait()
            return carry

        lax.fori_loop(0, NRND, round_body, 0, unroll=False)

        @pl.when(wid < NBLK - NFULL)
        def _():
            blk = NFULL + wid
            for k in range(4):
                pltpu.sync_copy(tables[k].at[:, pl.ds(blk * CHUNK, CHUNK)],
                                in_bufs[k].at[:, pl.ds(0, CHUNK)])
                transpose_compute(in_bufs[k], out_bufs[k], 1)
                pltpu.sync_copy(out_bufs[k].at[pl.ds(0, L), :],
                                outs[k].at[pl.ds(blk * L, L), :])

        @pl.when(wid == 0)
        def _():
            for tail, out in ((e0_hbm, o0_hbm), (e1_hbm, o1_hbm),
                              (e2_hbm, o2_hbm), (e3_hbm, o3_hbm)):
                pltpu.sync_copy(tail, tail_v)
                for p in range(TAIL // 8):
                    for q in range(8):
                        vec = plsc.load_gather(
                            tail_v,
                            [jnp.full((L,), 8 * p + q, jnp.int32), iota])
                        plsc.store_scatter(
                            out_v,
                            [jnp.full((L,), p, jnp.int32), q * L + iota],
                            vec)
                pltpu.sync_copy(
                    out_v.at[pl.ds(0, TAIL // 8)],
                    out.at[pl.ds(G - TAIL // 8, TAIL // 8), :])

    return convert_kernel


@functools.lru_cache(maxsize=None)
def _gather(B, D):
    assert D == L
    NW = NC * NS
    BPW = B // NW
    assert BPW % CHUNK == 0
    NCH = BPW // CHUNK

    mesh = plsc.VectorSubcoreMesh(
        core_axis_name="c", subcore_axis_name="s",
        num_cores=NC, num_subcores=NS)

    @functools.partial(
        pl.kernel,
        out_type=jax.ShapeDtypeStruct((B,), jnp.float32),
        mesh=mesh,
        compiler_params=pltpu.CompilerParams(needs_layout_passes=False),
        scratch_types=[
            pltpu.VMEM((BPW,), jnp.int32),
            pltpu.VMEM((BPW,), jnp.int32),
            pltpu.VMEM((BPW,), jnp.int32),
            pltpu.VMEM((BPW,), jnp.int32),
            pltpu.VMEM((CHUNK, 128), jnp.float32),
            pltpu.VMEM((CHUNK, 128), jnp.float32),
            pltpu.VMEM((CHUNK, 128), jnp.float32),
            pltpu.VMEM((CHUNK, 128), jnp.float32),
            pltpu.VMEM((BPW,), jnp.float32),
            pltpu.VMEM((BPW,), jnp.float32),
            pltpu.SemaphoreType.DMA,
            pltpu.SemaphoreType.DMA,
        ],
    )
    def gather_kernel(u_hbm, i_hbm, mod_u_hbm, mod_i_hbm, mem_u_hbm,
                      mem_i_hbm, alpha_hbm, out_hbm,
                      u_v, i_v, u8_v, i8_v, mu_v, mi_v, ku_v, ki_v,
                      a_v, o_v, sem, sem_a):
        wid = lax.axis_index("s") * NC + lax.axis_index("c")
        base = wid * BPW

        pltpu.sync_copy(u_hbm.at[pl.ds(base, BPW)], u_v)
        pltpu.sync_copy(i_hbm.at[pl.ds(base, BPW)], i_v)

        iota = lax.iota(jnp.int32, L)

        def shift_body(b, carry):
            s = pl.ds(b * L, L)
            u8_v[s] = lax.shift_right_logical(u_v[s], 3)
            i8_v[s] = lax.shift_right_logical(i_v[s], 3)
            return carry

        lax.fori_loop(0, BPW // L, shift_body, 0, unroll=False)

        a_copies = []
        for j in range(NCH):
            s = pl.ds(j * CHUNK, CHUNK)
            a_copies.append(
                pltpu.async_copy(alpha_hbm.at[u_v.at[s]], a_v.at[s], sem_a))
        for c in a_copies:
            c.wait()

        for j in range(NCH):
            s = pl.ds(j * CHUNK, CHUNK)
            copies = [
                pltpu.async_copy(mod_u_hbm.at[u8_v.at[s]], mu_v, sem),
                pltpu.async_copy(mod_i_hbm.at[i8_v.at[s]], mi_v, sem),
                pltpu.async_copy(mem_u_hbm.at[u8_v.at[s]], ku_v, sem),
                pltpu.async_copy(mem_i_hbm.at[i8_v.at[s]], ki_v, sem),
            ]
            for c in copies:
                c.wait()

            def blk_body(b, carry):
                pos = j * CHUNK + b * L
                rows = b * L + iota
                ucol = (u_v[pl.ds(pos, L)] & 7) * L
                icol = (i_v[pl.ds(pos, L)] & 7) * L
                acc1 = jnp.zeros((L,), jnp.float32)
                acc2 = jnp.zeros((L,), jnp.float32)
                for d in range(D):
                    acc1 = acc1 + (plsc.load_gather(mu_v, [rows, ucol + d])
                                   * plsc.load_gather(mi_v, [rows, icol + d]))
                    acc2 = acc2 + (plsc.load_gather(ku_v, [rows, ucol + d])
                                   * plsc.load_gather(ki_v, [rows, icol + d]))
                a = a_v[pl.ds(pos, L)]
                o_v[pl.ds(pos, L)] = a * acc1 + (1.0 - a) * acc2
                return carry

            lax.fori_loop(0, CHUNK // L, blk_body, 0, unroll=False)

        pltpu.sync_copy(o_v, out_hbm.at[pl.ds(base, BPW)])

    return gather_kernel


def kernel(user_indices, item_indices, mod_user_emb, mod_item_emb,
           mem_user_emb, mem_item_emb, alpha_table):
    B = user_indices.shape[0]
    N, D = mod_user_emb.shape
    tables = (mod_user_emb, mod_item_emb, mem_user_emb, mem_item_emb)
    tail0 = (N // 128) * 128
    conv = _convert(N, D)(
        *(t.T for t in tables), *(t[tail0:, :] for t in tables))
    return _gather(B, D)(user_indices, item_indices, *conv,
                         alpha_table.reshape(-1))
